# initial kernel scaffold (unmeasured)
import functools

import jax
import jax.numpy as jnp
from jax import lax
from jax.experimental import pallas as pl
from jax.experimental.pallas import tpu as pltpu

N_DEV = 4
GELU_C = 0.7978845608028654


def _gelu(y):
    return 0.5 * y * (1.0 + jnp.tanh(GELU_C * (y + 0.044715 * y * y * y)))


def kernel(x, w_mat):
    m_total, k_shard = x.shape
    k_total, n_total = w_mat.shape
    blk = m_total // N_DEV
    assert blk == k_shard
    n_chunk = 2048
    n_chunks = n_total // n_chunk

    def body(perm_ref, x_ref, w_ref, out_ref, comm_ref, send_sems, recv_sems):
        t = pl.program_id(0)
        me = lax.axis_index("i")

        def recv_desc(j):
            return pltpu.make_async_remote_copy(
                src_ref=comm_ref.at[pl.ds(j * blk, blk)],
                dst_ref=comm_ref.at[pl.ds(j * blk, blk)],
                send_sem=send_sems.at[0],
                recv_sem=recv_sems.at[j],
                device_id=(me,),
                device_id_type=pl.DeviceIdType.MESH,
            )

        @pl.when(t == 0)
        def _first_step():
            barrier_sem = pltpu.get_barrier_semaphore()
            for off in (1, 2, 3):
                pl.semaphore_signal(
                    barrier_sem,
                    inc=1,
                    device_id=((me + off) % N_DEV,),
                    device_id_type=pl.DeviceIdType.MESH,
                )
            pl.semaphore_wait(barrier_sem, N_DEV - 1)

            for h, off in enumerate((2, 1, 3)):
                dst = (me + off) % N_DEV
                rdma = pltpu.make_async_remote_copy(
                    src_ref=x_ref.at[pl.ds(dst * blk, blk)],
                    dst_ref=comm_ref.at[pl.ds(me * blk, blk)],
                    send_sem=send_sems.at[h],
                    recv_sem=recv_sems.at[me],
                    device_id=(dst,),
                    device_id_type=pl.DeviceIdType.MESH,
                )
                rdma.start()

            xblk = x_ref[pl.ds(me * blk, blk), :]
            for nc in range(n_chunks):
                sl = pl.ds(nc * n_chunk, n_chunk)
                out_ref[:, sl] = jnp.dot(
                    xblk, w_ref[:, sl], preferred_element_type=jnp.float32
                )

        @pl.when((t == 1) | (t == 2))
        def _mid_steps():
            j = perm_ref[t]
            recv_desc(j).wait_recv()
            xblk = comm_ref[pl.ds(j * blk, blk), :]
            for nc in range(n_chunks):
                sl = pl.ds(nc * n_chunk, n_chunk)
                out_ref[:, sl] += jnp.dot(
                    xblk, w_ref[:, sl], preferred_element_type=jnp.float32
                )

        @pl.when(t == N_DEV - 1)
        def _last_step():
            j = perm_ref[t]
            recv_desc(j).wait_recv()
            xblk = comm_ref[pl.ds(j * blk, blk), :]
            for nc in range(n_chunks):
                sl = pl.ds(nc * n_chunk, n_chunk)
                acc = out_ref[:, sl] + jnp.dot(
                    xblk, w_ref[:, sl], preferred_element_type=jnp.float32
                )
                out_ref[:, sl] = _gelu(acc)
            for h in range(N_DEV - 1):
                snd = pltpu.make_async_remote_copy(
                    src_ref=x_ref.at[pl.ds(0, blk)],
                    dst_ref=comm_ref.at[pl.ds(0, blk)],
                    send_sem=send_sems.at[h],
                    recv_sem=recv_sems.at[me],
                    device_id=(me,),
                    device_id_type=pl.DeviceIdType.MESH,
                )
                snd.wait_send()

    me = lax.axis_index("i")
    perm = (me + jnp.array([0, 1, 3, 2], dtype=jnp.int32)) % N_DEV

    grid_spec = pltpu.PrefetchScalarGridSpec(
        num_scalar_prefetch=1,
        grid=(N_DEV,),
        in_specs=[
            pl.BlockSpec((m_total, k_shard), lambda t, perm: (0, 0)),
            pl.BlockSpec((blk, n_total), lambda t, perm: (perm[t], 0)),
        ],
        out_specs=pl.BlockSpec((blk, n_total), lambda t, perm: (0, 0)),
        scratch_shapes=[
            pltpu.VMEM((m_total, k_shard), jnp.bfloat16),
            pltpu.SemaphoreType.DMA((N_DEV - 1,)),
            pltpu.SemaphoreType.DMA((N_DEV,)),
        ],
    )

    return pl.pallas_call(
        body,
        grid_spec=grid_spec,
        out_shape=jax.ShapeDtypeStruct((blk, n_total), jnp.float32),
        compiler_params=pltpu.CompilerParams(
            collective_id=0,
            dimension_semantics=("arbitrary",),
        ),
    )(perm, x, w_mat)


# baseline (device time: 162782 ns/iter reference)
import jax
import jax.numpy as jnp
from jax import lax
from jax.experimental import pallas as pl
from jax.experimental.pallas import tpu as pltpu

N_DEV = 4
GELU_C = 0.7978845608028654


def _gelu(y):
    return 0.5 * y * (1.0 + jnp.tanh(GELU_C * (y + 0.044715 * y * y * y)))


def kernel(x, w_mat):
    m_total, k_shard = x.shape
    k_total, n_total = w_mat.shape
    blk = m_total // N_DEV
    assert blk == k_shard
    n_half = n_total // 2
    nch = 1024
    n_nc = n_half // nch
    n_chunks = 2 * N_DEV * n_nc

    def body(x_hbm, w_hbm, out_hbm, acc, xbf, comm, wstage, xstage,
             wsems, xsems, out_sems, send_sems, recv_sems):
        me = lax.axis_index("i")
        perm = [me, (me + 1) % N_DEV, (me + 3) % N_DEV, (me + 2) % N_DEV]

        barrier_sem = pltpu.get_barrier_semaphore()
        for off in (1, 2, 3):
            pl.semaphore_signal(
                barrier_sem,
                inc=1,
                device_id=((me + off) % N_DEV,),
                device_id_type=pl.DeviceIdType.MESH,
            )
        pl.semaphore_wait(barrier_sem, N_DEV - 1)

        def xdma(c, slot):
            return pltpu.make_async_copy(
                x_hbm.at[pl.ds(c * blk, blk)], xstage.at[slot],
                xsems.at[slot],
            )

        xdma(0, 0).start()
        for c in range(N_DEV):
            xdma(c, c % 2).wait()
            if c + 1 < N_DEV:
                xdma(c + 1, (c + 1) % 2).start()
            xbf[pl.ds(c * blk, blk), :] = xstage[c % 2].astype(jnp.bfloat16)

        sends = []
        for h, off in enumerate((2, 1, 3)):
            dst = (me + off) % N_DEV
            rdma = pltpu.make_async_remote_copy(
                src_ref=xbf.at[pl.ds(dst * blk, blk)],
                dst_ref=comm.at[pl.ds(me * blk, blk)],
                send_sem=send_sems.at[h],
                recv_sem=recv_sems.at[me],
                device_id=(dst,),
                device_id_type=pl.DeviceIdType.MESH,
            )
            rdma.start()
            sends.append(rdma)

        def wdma(c, slot):
            p, r = divmod(c, N_DEV * n_nc)
            kt, nc = divmod(r, n_nc)
            return pltpu.make_async_copy(
                w_hbm.at[pl.ds(perm[kt] * blk, blk),
                         pl.ds(p * n_half + nc * nch, nch)],
                wstage.at[slot],
                wsems.at[slot],
            )

        def outdma(p, nc):
            return pltpu.make_async_copy(
                acc.at[:, pl.ds(nc * nch, nch)],
                out_hbm.at[:, pl.ds(p * n_half + nc * nch, nch)],
                out_sems.at[nc],
            )

        wdma(0, 0).start()
        for c in range(n_chunks):
            p, r = divmod(c, N_DEV * n_nc)
            kt, nc = divmod(r, n_nc)
            slot = c % 2
            wdma(c, slot).wait()
            if c + 1 < n_chunks:
                wdma(c + 1, (c + 1) % 2).start()
            j = perm[kt]
            if p == 0 and kt > 0 and nc == 0:
                rec = pltpu.make_async_remote_copy(
                    src_ref=comm.at[pl.ds(j * blk, blk)],
                    dst_ref=comm.at[pl.ds(j * blk, blk)],
                    send_sem=send_sems.at[0],
                    recv_sem=recv_sems.at[j],
                    device_id=(me,),
                    device_id_type=pl.DeviceIdType.MESH,
                )
                rec.wait_recv()
            if p == 1 and kt == 0:
                outdma(0, nc).wait()
            src = xbf if kt == 0 else comm
            xblk = src[pl.ds(j * blk, blk), :]
            wbf = wstage[slot].astype(jnp.bfloat16)
            sl = pl.ds(nc * nch, nch)
            contrib = jnp.dot(xblk, wbf, preferred_element_type=jnp.float32)
            if kt == 0:
                acc[:, sl] = contrib
            elif kt < N_DEV - 1:
                acc[:, sl] += contrib
            else:
                acc[:, sl] = _gelu(acc[:, sl] + contrib)
                outdma(p, nc).start()

        for nc in range(n_nc):
            outdma(1, nc).wait()
        for rdma in sends:
            rdma.wait_send()

    return pl.pallas_call(
        body,
        out_shape=jax.ShapeDtypeStruct((blk, n_total), jnp.float32),
        in_specs=[
            pl.BlockSpec(memory_space=pltpu.MemorySpace.HBM),
            pl.BlockSpec(memory_space=pltpu.MemorySpace.HBM),
        ],
        out_specs=pl.BlockSpec(memory_space=pltpu.MemorySpace.HBM),
        scratch_shapes=[
            pltpu.VMEM((blk, n_half), jnp.float32),
            pltpu.VMEM((m_total, k_shard), jnp.bfloat16),
            pltpu.VMEM((m_total, k_shard), jnp.bfloat16),
            pltpu.VMEM((2, blk, nch), jnp.float32),
            pltpu.VMEM((2, blk, k_shard), jnp.float32),
            pltpu.SemaphoreType.DMA((2,)),
            pltpu.SemaphoreType.DMA((2,)),
            pltpu.SemaphoreType.DMA((n_nc,)),
            pltpu.SemaphoreType.DMA((N_DEV - 1,)),
            pltpu.SemaphoreType.DMA((N_DEV,)),
        ],
        compiler_params=pltpu.CompilerParams(
            collective_id=0,
            vmem_limit_bytes=60 * 1024 * 1024,
        ),
    )(x, w_mat)


# device time: 152776 ns/iter; 1.0655x vs baseline; 1.0655x over previous
import jax
import jax.numpy as jnp
from jax import lax
from jax.experimental import pallas as pl
from jax.experimental.pallas import tpu as pltpu

N_DEV = 4
N_SLOT = 4
GELU_C = 0.7978845608028654


def _gelu(y):
    return 0.5 * y * (1.0 + jnp.tanh(GELU_C * (y + 0.044715 * y * y * y)))


def kernel(x, w_mat):
    m_total, k_shard = x.shape
    k_total, n_total = w_mat.shape
    blk = m_total // N_DEV
    assert blk == k_shard
    n_half = n_total // 2
    nch = 1024
    n_nc = n_half // nch
    n_chunks = 2 * N_DEV * n_nc

    def body(x_hbm, w_hbm, out_hbm, acc, xbf, comm, stage,
             ssems, out_sems, send_sems, recv_sems):
        me = lax.axis_index("i")
        perm = [me, (me + 1) % N_DEV, (me + 3) % N_DEV, (me + 2) % N_DEV]

        barrier_sem = pltpu.get_barrier_semaphore()
        for off in (1, 2, 3):
            pl.semaphore_signal(
                barrier_sem,
                inc=1,
                device_id=((me + off) % N_DEV,),
                device_id_type=pl.DeviceIdType.MESH,
            )
        pl.semaphore_wait(barrier_sem, N_DEV - 1)

        xorder = [(me + 2) % N_DEV, (me + 1) % N_DEV, (me + 3) % N_DEV, me]
        for h, src_blk in enumerate(xorder):
            pltpu.make_async_copy(
                x_hbm.at[pl.ds(src_blk * blk, blk)], stage.at[h],
                ssems.at[h],
            ).start()
        sends = []
        for h, src_blk in enumerate(xorder):
            pltpu.make_async_copy(
                x_hbm.at[pl.ds(src_blk * blk, blk)], stage.at[h],
                ssems.at[h],
            ).wait()
            xbf[pl.ds(src_blk * blk, blk), :] = stage[h].astype(jnp.bfloat16)
            if h < 3:
                rdma = pltpu.make_async_remote_copy(
                    src_ref=xbf.at[pl.ds(src_blk * blk, blk)],
                    dst_ref=comm.at[pl.ds(me * blk, blk)],
                    send_sem=send_sems.at[h],
                    recv_sem=recv_sems.at[me],
                    device_id=(src_blk,),
                    device_id_type=pl.DeviceIdType.MESH,
                )
                rdma.start()
                sends.append(rdma)

        def wdma(c, slot):
            p, r = divmod(c, N_DEV * n_nc)
            kt, nc = divmod(r, n_nc)
            return pltpu.make_async_copy(
                w_hbm.at[pl.ds(perm[kt] * blk, blk),
                         pl.ds(p * n_half + nc * nch, nch)],
                stage.at[slot],
                ssems.at[slot],
            )

        def outdma(p, nc):
            return pltpu.make_async_copy(
                acc.at[:, pl.ds(nc * nch, nch)],
                out_hbm.at[:, pl.ds(p * n_half + nc * nch, nch)],
                out_sems.at[nc],
            )

        for c0 in range(N_SLOT - 1):
            wdma(c0, c0 % N_SLOT).start()
        for c in range(n_chunks):
            p, r = divmod(c, N_DEV * n_nc)
            kt, nc = divmod(r, n_nc)
            slot = c % N_SLOT
            wdma(c, slot).wait()
            if c + N_SLOT - 1 < n_chunks:
                wdma(c + N_SLOT - 1, (c + N_SLOT - 1) % N_SLOT).start()
            j = perm[kt]
            if p == 0 and kt > 0 and nc == 0:
                rec = pltpu.make_async_remote_copy(
                    src_ref=comm.at[pl.ds(j * blk, blk)],
                    dst_ref=comm.at[pl.ds(j * blk, blk)],
                    send_sem=send_sems.at[0],
                    recv_sem=recv_sems.at[j],
                    device_id=(me,),
                    device_id_type=pl.DeviceIdType.MESH,
                )
                rec.wait_recv()
            if p == 1 and kt == 0:
                outdma(0, nc).wait()
            src = xbf if kt == 0 else comm
            xblk = src[pl.ds(j * blk, blk), :]
            wbf = stage[slot].astype(jnp.bfloat16)
            sl = pl.ds(nc * nch, nch)
            contrib = jnp.dot(xblk, wbf, preferred_element_type=jnp.float32)
            if kt == 0:
                acc[:, sl] = contrib
            elif kt < N_DEV - 1:
                acc[:, sl] += contrib
            else:
                acc[:, sl] = _gelu(acc[:, sl] + contrib)
                outdma(p, nc).start()

        for nc in range(n_nc):
            outdma(1, nc).wait()
        for rdma in sends:
            rdma.wait_send()

    return pl.pallas_call(
        body,
        out_shape=jax.ShapeDtypeStruct((blk, n_total), jnp.float32),
        in_specs=[
            pl.BlockSpec(memory_space=pltpu.MemorySpace.HBM),
            pl.BlockSpec(memory_space=pltpu.MemorySpace.HBM),
        ],
        out_specs=pl.BlockSpec(memory_space=pltpu.MemorySpace.HBM),
        scratch_shapes=[
            pltpu.VMEM((blk, n_half), jnp.float32),
            pltpu.VMEM((m_total, k_shard), jnp.bfloat16),
            pltpu.VMEM((m_total, k_shard), jnp.bfloat16),
            pltpu.VMEM((N_SLOT, blk, nch), jnp.float32),
            pltpu.SemaphoreType.DMA((N_SLOT,)),
            pltpu.SemaphoreType.DMA((n_nc,)),
            pltpu.SemaphoreType.DMA((N_DEV - 1,)),
            pltpu.SemaphoreType.DMA((N_DEV,)),
        ],
        compiler_params=pltpu.CompilerParams(
            collective_id=0,
            vmem_limit_bytes=60 * 1024 * 1024,
        ),
    )(x, w_mat)


# device time: 147793 ns/iter; 1.1014x vs baseline; 1.0337x over previous
import jax
import jax.numpy as jnp
from jax import lax
from jax.experimental import pallas as pl
from jax.experimental.pallas import tpu as pltpu

N_DEV = 4
N_SLOT = 4
GELU_C = 0.7978845608028654


def _gelu(y):
    return 0.5 * y * (1.0 + jnp.tanh(GELU_C * (y + 0.044715 * y * y * y)))


def kernel(x, w_mat):
    m_total, k_shard = x.shape
    k_total, n_total = w_mat.shape
    blk = m_total // N_DEV
    assert blk == k_shard
    nch = 512
    n_nc = n_total // nch
    n_chunks = N_DEV * n_nc
    xh = blk // 2

    def body(x_hbm, w_hbm, out_hbm, acc, xbf, comm, stage,
             ssems, out_sems, send_sems, recv_sems):
        me = lax.axis_index("i")
        perm = [me, (me + 1) % N_DEV, (me + 3) % N_DEV, (me + 2) % N_DEV]

        barrier_sem = pltpu.get_barrier_semaphore()
        for off in (1, 2, 3):
            pl.semaphore_signal(
                barrier_sem,
                inc=1,
                device_id=((me + off) % N_DEV,),
                device_id_type=pl.DeviceIdType.MESH,
            )
        pl.semaphore_wait(barrier_sem, N_DEV - 1)

        xorder = [(me + 2) % N_DEV, (me + 1) % N_DEV, (me + 3) % N_DEV, me]

        def xdma(i, slot):
            b, half = divmod(i, 2)
            return pltpu.make_async_copy(
                x_hbm.at[pl.ds(xorder[b] * blk, blk), pl.ds(half * xh, xh)],
                stage.at[slot],
                ssems.at[slot],
            )

        for i in range(N_SLOT):
            xdma(i, i).start()
        sends = []
        for i in range(2 * N_DEV):
            b, half = divmod(i, 2)
            slot = i % N_SLOT
            xdma(i, slot).wait()
            xbf[pl.ds(xorder[b] * blk, blk), pl.ds(half * xh, xh)] = (
                stage[slot].astype(jnp.bfloat16))
            if i + N_SLOT < 2 * N_DEV:
                xdma(i + N_SLOT, slot).start()
            if half == 1 and b < 3:
                rdma = pltpu.make_async_remote_copy(
                    src_ref=xbf.at[pl.ds(xorder[b] * blk, blk)],
                    dst_ref=comm.at[pl.ds(me * blk, blk)],
                    send_sem=send_sems.at[b],
                    recv_sem=recv_sems.at[me],
                    device_id=(xorder[b],),
                    device_id_type=pl.DeviceIdType.MESH,
                )
                rdma.start()
                sends.append(rdma)

        def wdma(c, slot):
            kt, cc = divmod(c, n_nc)
            return pltpu.make_async_copy(
                w_hbm.at[pl.ds(perm[kt] * blk, blk), pl.ds(cc * nch, nch)],
                stage.at[slot],
                ssems.at[slot],
            )

        def outdma(cc):
            return pltpu.make_async_copy(
                acc.at[:, pl.ds(cc * nch, nch)],
                out_hbm.at[:, pl.ds(cc * nch, nch)],
                out_sems.at[cc],
            )

        for c0 in range(N_SLOT - 1):
            wdma(c0, c0 % N_SLOT).start()
        for c in range(n_chunks):
            kt, cc = divmod(c, n_nc)
            slot = c % N_SLOT
            wdma(c, slot).wait()
            if c + N_SLOT - 1 < n_chunks:
                wdma(c + N_SLOT - 1, (c + N_SLOT - 1) % N_SLOT).start()
            j = perm[kt]
            if kt > 0 and cc == 0:
                rec = pltpu.make_async_remote_copy(
                    src_ref=comm.at[pl.ds(j * blk, blk)],
                    dst_ref=comm.at[pl.ds(j * blk, blk)],
                    send_sem=send_sems.at[0],
                    recv_sem=recv_sems.at[j],
                    device_id=(me,),
                    device_id_type=pl.DeviceIdType.MESH,
                )
                rec.wait_recv()
            src = xbf if kt == 0 else comm
            xblk = src[pl.ds(j * blk, blk), :]
            wbf = stage[slot].astype(jnp.bfloat16)
            sl = pl.ds(cc * nch, nch)
            contrib = jnp.dot(xblk, wbf, preferred_element_type=jnp.float32)
            if kt == 0:
                acc[:, sl] = contrib
            elif kt < N_DEV - 1:
                acc[:, sl] += contrib
            else:
                acc[:, sl] = _gelu(acc[:, sl] + contrib)
                outdma(cc).start()

        for cc in range(n_nc):
            outdma(cc).wait()
        for rdma in sends:
            rdma.wait_send()

    return pl.pallas_call(
        body,
        out_shape=jax.ShapeDtypeStruct((blk, n_total), jnp.float32),
        in_specs=[
            pl.BlockSpec(memory_space=pltpu.MemorySpace.HBM),
            pl.BlockSpec(memory_space=pltpu.MemorySpace.HBM),
        ],
        out_specs=pl.BlockSpec(memory_space=pltpu.MemorySpace.HBM),
        scratch_shapes=[
            pltpu.VMEM((blk, n_total), jnp.float32),
            pltpu.VMEM((m_total, k_shard), jnp.bfloat16),
            pltpu.VMEM((m_total, k_shard), jnp.bfloat16),
            pltpu.VMEM((N_SLOT, blk, nch), jnp.float32),
            pltpu.SemaphoreType.DMA((N_SLOT,)),
            pltpu.SemaphoreType.DMA((n_nc,)),
            pltpu.SemaphoreType.DMA((N_DEV - 1,)),
            pltpu.SemaphoreType.DMA((N_DEV,)),
        ],
        compiler_params=pltpu.CompilerParams(
            collective_id=0,
            vmem_limit_bytes=63 * 1024 * 1024,
        ),
    )(x, w_mat)


# device time: 131870 ns/iter; 1.2344x vs baseline; 1.1207x over previous
import jax
import jax.numpy as jnp
from jax import lax
from jax.experimental import pallas as pl
from jax.experimental.pallas import tpu as pltpu

N_DEV = 4
N_SLOT = 4
GELU_C = 0.7978845608028654


def _gelu(y):
    return 0.5 * y * (1.0 + jnp.tanh(GELU_C * (y + 0.044715 * y * y * y)))


def kernel(x, w_mat):
    m_total, k_shard = x.shape
    k_total, n_total = w_mat.shape
    blk = m_total // N_DEV
    assert blk == k_shard
    nch = 512
    n_nc = n_total // nch
    n_chunks = N_DEV * n_nc
    xh = blk // 2

    def body(x_hbm, w_hbm, out_hbm, acc, xbf, comm, stage,
             ssems, out_sems, send_sems, recv_sems):
        me = lax.axis_index("i")
        perm = [me, (me + 1) % N_DEV, (me + 3) % N_DEV, (me + 2) % N_DEV]

        barrier_sem = pltpu.get_barrier_semaphore()
        for off in (1, 2, 3):
            pl.semaphore_signal(
                barrier_sem,
                inc=1,
                device_id=((me + off) % N_DEV,),
                device_id_type=pl.DeviceIdType.MESH,
            )
        pl.semaphore_wait(barrier_sem, N_DEV - 1)

        xorder = [(me + 1) % N_DEV, (me + 3) % N_DEV, (me + 2) % N_DEV, me]

        def xdma(i, slot):
            b, half = divmod(i, 2)
            return pltpu.make_async_copy(
                x_hbm.at[pl.ds(xorder[b] * blk, blk), pl.ds(half * xh, xh)],
                stage.at[slot],
                ssems.at[slot],
            )

        for i in range(N_SLOT):
            xdma(i, i).start()
        sends = []
        for i in range(2 * N_DEV):
            b, half = divmod(i, 2)
            slot = i % N_SLOT
            xdma(i, slot).wait()
            xbf[pl.ds(xorder[b] * blk, blk), pl.ds(half * xh, xh)] = (
                stage[slot].astype(jnp.bfloat16))
            if i + N_SLOT < 2 * N_DEV:
                xdma(i + N_SLOT, slot).start()
            if half == 1 and b < 3:
                rdma = pltpu.make_async_remote_copy(
                    src_ref=xbf.at[pl.ds(xorder[b] * blk, blk)],
                    dst_ref=comm.at[pl.ds(me * blk, blk)],
                    send_sem=send_sems.at[b],
                    recv_sem=recv_sems.at[me],
                    device_id=(xorder[b],),
                    device_id_type=pl.DeviceIdType.MESH,
                )
                if b < 2:
                    rdma.start()
                sends.append(rdma)

        def wdma(c, slot):
            kt, cc = divmod(c, n_nc)
            return pltpu.make_async_copy(
                w_hbm.at[pl.ds(perm[kt] * blk, blk), pl.ds(cc * nch, nch)],
                stage.at[slot],
                ssems.at[slot],
            )

        def outdma(cc):
            return pltpu.make_async_copy(
                acc.at[:, pl.ds(cc * nch, nch)],
                out_hbm.at[:, pl.ds(cc * nch, nch)],
                out_sems.at[cc],
            )

        for c0 in range(N_SLOT - 1):
            wdma(c0, c0 % N_SLOT).start()
        for c in range(n_chunks):
            kt, cc = divmod(c, n_nc)
            slot = c % N_SLOT
            wdma(c, slot).wait()
            if c + N_SLOT - 1 < n_chunks:
                wdma(c + N_SLOT - 1, (c + N_SLOT - 1) % N_SLOT).start()
            if c == 12:
                sends[0].wait_send()
                sends[1].wait_send()
                sends[2].start()
            j = perm[kt]
            if kt > 0 and cc == 0:
                rec = pltpu.make_async_remote_copy(
                    src_ref=comm.at[pl.ds(j * blk, blk)],
                    dst_ref=comm.at[pl.ds(j * blk, blk)],
                    send_sem=send_sems.at[0],
                    recv_sem=recv_sems.at[j],
                    device_id=(me,),
                    device_id_type=pl.DeviceIdType.MESH,
                )
                rec.wait_recv()
            src = xbf if kt == 0 else comm
            xblk = src[pl.ds(j * blk, blk), :]
            wbf = stage[slot].astype(jnp.bfloat16)
            sl = pl.ds(cc * nch, nch)
            contrib = jnp.dot(xblk, wbf, preferred_element_type=jnp.float32)
            if kt == 0:
                acc[:, sl] = contrib
            elif kt < N_DEV - 1:
                acc[:, sl] += contrib
            else:
                acc[:, sl] = _gelu(acc[:, sl] + contrib)
                outdma(cc).start()

        for cc in range(n_nc):
            outdma(cc).wait()
        sends[2].wait_send()

    return pl.pallas_call(
        body,
        out_shape=jax.ShapeDtypeStruct((blk, n_total), jnp.float32),
        in_specs=[
            pl.BlockSpec(memory_space=pltpu.MemorySpace.HBM),
            pl.BlockSpec(memory_space=pltpu.MemorySpace.HBM),
        ],
        out_specs=pl.BlockSpec(memory_space=pltpu.MemorySpace.HBM),
        scratch_shapes=[
            pltpu.VMEM((blk, n_total), jnp.float32),
            pltpu.VMEM((m_total, k_shard), jnp.bfloat16),
            pltpu.VMEM((m_total, k_shard), jnp.bfloat16),
            pltpu.VMEM((N_SLOT, blk, nch), jnp.float32),
            pltpu.SemaphoreType.DMA((N_SLOT,)),
            pltpu.SemaphoreType.DMA((n_nc,)),
            pltpu.SemaphoreType.DMA((N_DEV - 1,)),
            pltpu.SemaphoreType.DMA((N_DEV,)),
        ],
        compiler_params=pltpu.CompilerParams(
            collective_id=0,
            vmem_limit_bytes=63 * 1024 * 1024,
        ),
    )(x, w_mat)


# device time: 131587 ns/iter; 1.2371x vs baseline; 1.0022x over previous
import jax
import jax.numpy as jnp
from jax import lax
from jax.experimental import pallas as pl
from jax.experimental.pallas import tpu as pltpu

N_DEV = 4
N_SLOT = 4
GELU_C = 0.7978845608028654


def _gelu(y):
    return 0.5 * y * (1.0 + jnp.tanh(GELU_C * (y + 0.044715 * y * y * y)))


def kernel(x, w_mat):
    m_total, k_shard = x.shape
    k_total, n_total = w_mat.shape
    blk = m_total // N_DEV
    assert blk == k_shard
    nch = 512
    n_nc = n_total // nch
    n_chunks = N_DEV * n_nc
    xh = blk // 2

    def body(x_hbm, w_hbm, out_hbm, acc, xbf, comm, stage,
             ssems, out_sems, send_sems, recv_sems):
        me = lax.axis_index("i")
        perm = [me, (me + 1) % N_DEV, (me + 3) % N_DEV, (me + 2) % N_DEV]

        barrier_sem = pltpu.get_barrier_semaphore()
        for off in (1, 2, 3):
            pl.semaphore_signal(
                barrier_sem,
                inc=1,
                device_id=((me + off) % N_DEV,),
                device_id_type=pl.DeviceIdType.MESH,
            )
        pl.semaphore_wait(barrier_sem, N_DEV - 1)

        xorder = [(me + 1) % N_DEV, (me + 3) % N_DEV, (me + 2) % N_DEV, me]

        def xdma(i, slot):
            b, half = divmod(i, 2)
            return pltpu.make_async_copy(
                x_hbm.at[pl.ds(xorder[b] * blk, blk), pl.ds(half * xh, xh)],
                stage.at[slot],
                ssems.at[slot],
            )

        for i in range(N_SLOT):
            xdma(i, i).start()
        sends = []
        for i in range(2 * N_DEV):
            b, half = divmod(i, 2)
            slot = i % N_SLOT
            xdma(i, slot).wait()
            xbf[pl.ds(xorder[b] * blk, blk), pl.ds(half * xh, xh)] = (
                stage[slot].astype(jnp.bfloat16))
            if i + N_SLOT < 2 * N_DEV:
                xdma(i + N_SLOT, slot).start()
            if half == 1 and b < 3:
                rdma = pltpu.make_async_remote_copy(
                    src_ref=xbf.at[pl.ds(xorder[b] * blk, blk)],
                    dst_ref=comm.at[pl.ds(me * blk, blk)],
                    send_sem=send_sems.at[b],
                    recv_sem=recv_sems.at[me],
                    device_id=(xorder[b],),
                    device_id_type=pl.DeviceIdType.MESH,
                )
                if b < 2:
                    rdma.start()
                sends.append(rdma)

        def wdma(c, slot):
            kt, cc = divmod(c, n_nc)
            return pltpu.make_async_copy(
                w_hbm.at[pl.ds(perm[kt] * blk, blk), pl.ds(cc * nch, nch)],
                stage.at[slot],
                ssems.at[slot],
            )

        def outdma(cc):
            return pltpu.make_async_copy(
                acc.at[:, pl.ds(cc * nch, nch)],
                out_hbm.at[:, pl.ds(cc * nch, nch)],
                out_sems.at[cc],
            )

        for c0 in range(N_SLOT - 1):
            wdma(c0, c0 % N_SLOT).start()
        for c in range(n_chunks):
            kt, cc = divmod(c, n_nc)
            slot = c % N_SLOT
            wdma(c, slot).wait()
            if c + N_SLOT - 1 < n_chunks:
                wdma(c + N_SLOT - 1, (c + N_SLOT - 1) % N_SLOT).start()
            if c == 12:
                sends[0].wait_send()
                sends[1].wait_send()
                sends[2].start()
            j = perm[kt]
            if kt > 0 and cc == 0:
                rec = pltpu.make_async_remote_copy(
                    src_ref=comm.at[pl.ds(j * blk, blk)],
                    dst_ref=comm.at[pl.ds(j * blk, blk)],
                    send_sem=send_sems.at[0],
                    recv_sem=recv_sems.at[j],
                    device_id=(me,),
                    device_id_type=pl.DeviceIdType.MESH,
                )
                rec.wait_recv()
            src = xbf if kt == 0 else comm
            xblk = src[pl.ds(j * blk, blk), :]
            wbf = stage[slot].astype(jnp.bfloat16)
            sl = pl.ds(cc * nch, nch)
            contrib = jnp.dot(xblk, wbf, preferred_element_type=jnp.float32)
            if kt == 0:
                acc[:, sl] = contrib
            elif kt < N_DEV - 1:
                acc[:, sl] += contrib
            else:
                acc[:, sl] = _gelu(acc[:, sl] + contrib)
                outdma(cc).start()

        for cc in range(n_nc):
            outdma(cc).wait()
        sends[2].wait_send()

    return pl.pallas_call(
        body,
        out_shape=jax.ShapeDtypeStruct((blk, n_total), jnp.float32),
        in_specs=[
            pl.BlockSpec(memory_space=pltpu.MemorySpace.HBM),
            pl.BlockSpec(memory_space=pltpu.MemorySpace.HBM),
        ],
        out_specs=pl.BlockSpec(memory_space=pltpu.MemorySpace.HBM),
        scratch_shapes=[
            pltpu.VMEM((blk, n_total), jnp.float32),
            pltpu.VMEM((m_total, k_shard), jnp.bfloat16),
            pltpu.VMEM((m_total, k_shard), jnp.bfloat16),
            pltpu.VMEM((N_SLOT, blk, nch), jnp.float32),
            pltpu.SemaphoreType.DMA((N_SLOT,)),
            pltpu.SemaphoreType.DMA((n_nc,)),
            pltpu.SemaphoreType.DMA((N_DEV - 1,)),
            pltpu.SemaphoreType.DMA((N_DEV,)),
        ],
        compiler_params=pltpu.CompilerParams(
            collective_id=0,
            vmem_limit_bytes=63 * 1024 * 1024,
            skip_device_barrier=True,
        ),
    )(x, w_mat)
